# Initial kernel scaffold; baseline (speedup 1.0000x reference)
import jax
import jax.numpy as jnp
from jax.experimental import pallas as pl


def kernel(x, edge_index, W_lin, b_lin, Wl1, bl1, Wr1, Wl2, bl2, Wr2):
    raise NotImplementedError("write your pallas kernel here")


# speedup vs baseline: 7.3269x; 7.3269x over previous
import functools

import jax
import jax.numpy as jnp
from jax import lax
from jax.experimental import pallas as pl
from jax.experimental.pallas import tpu as pltpu
from jax.experimental.pallas import tpu_sc as plsc

NC = 2
NS = 16
NW = NC * NS
LANES = 16
C = 128


def _dg(a, b):
  return lax.dot_general(a, b, (((1,), (1,)), ((), ())),
                         preferred_element_type=jnp.float32)


@functools.lru_cache(maxsize=None)
def _make_scatter(n_chunks, d, nr):
  rpt = nr // NS
  n_out_blk = rpt // C

  mesh = plsc.VectorSubcoreMesh(core_axis_name="c", subcore_axis_name="s")
  out_type = jax.ShapeDtypeStruct((NC, nr, d), jnp.float32)
  scratch = [
      pltpu.VMEM((n_chunks, C), jnp.int32),
      pltpu.VMEM((n_chunks, C), jnp.int32),
      pltpu.VMEM((C, d), jnp.float32),
      pltpu.VMEM_SHARED((nr, d), jnp.float32),
      pltpu.SemaphoreType.DMA,
  ]

  def body(table, srcs, dsts, zeros, out_acc, srcv, dstv, rowbuf, acc_sh, sem):
    cid = lax.axis_index("c")
    sid = lax.axis_index("s")
    wid = sid * NC + cid

    pltpu.sync_copy(srcs.at[wid], srcv)
    pltpu.sync_copy(dsts.at[wid], dstv)
    pltpu.sync_copy(zeros, rowbuf)
    for b in range(n_out_blk):
      pltpu.sync_copy(rowbuf, acc_sh.at[pl.ds(sid * rpt + b * C, C)])

    plsc.subcore_barrier()

    def chunk(j, carry):
      pltpu.async_copy(table.at[srcv.at[j]], rowbuf, sem).wait()
      pltpu.sync_copy(rowbuf, acc_sh.at[dstv.at[j]], add=True)
      return carry

    lax.fori_loop(0, n_chunks, chunk, 0)

    plsc.subcore_barrier()

    for b in range(n_out_blk):
      sl = pl.ds(sid * rpt + b * C, C)
      pltpu.sync_copy(acc_sh.at[sl], rowbuf)
      pltpu.sync_copy(rowbuf, out_acc.at[cid, sl])

  return pl.kernel(body, out_type=out_type, mesh=mesh, scratch_types=scratch)


@functools.lru_cache(maxsize=None)
def _make_count(n_chunks, nr, w):
  rpt = nr // NS
  n_out_blk = rpt // C

  mesh = plsc.VectorSubcoreMesh(core_axis_name="c", subcore_axis_name="s")
  out_type = jax.ShapeDtypeStruct((NC, nr, w), jnp.float32)
  scratch = [
      pltpu.VMEM((n_chunks, C), jnp.int32),
      pltpu.VMEM((C, w), jnp.float32),
      pltpu.VMEM((C, w), jnp.float32),
      pltpu.VMEM_SHARED((nr, w), jnp.float32),
  ]

  def body(dsts, zeros, ones, out_cnt, dstv, ov, zc, cnt_sh):
    cid = lax.axis_index("c")
    sid = lax.axis_index("s")
    wid = sid * NC + cid

    pltpu.sync_copy(dsts.at[wid], dstv)
    pltpu.sync_copy(ones, ov)
    pltpu.sync_copy(zeros, zc)
    for b in range(n_out_blk):
      pltpu.sync_copy(zc, cnt_sh.at[pl.ds(sid * rpt + b * C, C)])

    plsc.subcore_barrier()

    def chunk(j, carry):
      pltpu.sync_copy(ov, cnt_sh.at[dstv.at[j]], add=True)
      return carry

    lax.fori_loop(0, n_chunks, chunk, 0)

    plsc.subcore_barrier()

    for b in range(n_out_blk):
      sl = pl.ds(sid * rpt + b * C, C)
      pltpu.sync_copy(cnt_sh.at[sl], zc)
      pltpu.sync_copy(zc, out_cnt.at[cid, sl])

  return pl.kernel(body, out_type=out_type, mesh=mesh, scratch_types=scratch)


def _stage_pre(x, w_lin, b_lin, wl1, bl1, wr1, blk):
  nr, d_in = x.shape
  d_h = w_lin.shape[0]
  grid = nr // blk
  full = lambda r, c: pl.BlockSpec((r, c), lambda i: (0, 0))

  def body(x_r, wlin_r, blin_r, wl1_r, bl1_r, wr1_r, g1_r, r1_r):
    h = jnp.maximum(_dg(x_r[...], wlin_r[...]) + blin_r[...], 0.0)
    g1_r[...] = _dg(h, wl1_r[...])
    r1_r[...] = _dg(h, wr1_r[...]) + bl1_r[...]

  return pl.pallas_call(
      body,
      grid=(grid,),
      in_specs=[
          pl.BlockSpec((blk, d_in), lambda i: (i, 0)),
          full(d_h, d_in), full(1, d_h),
          full(d_h, d_h), full(1, d_h), full(d_h, d_h),
      ],
      out_specs=[pl.BlockSpec((blk, d_h), lambda i: (i, 0))] * 2,
      out_shape=[jax.ShapeDtypeStruct((nr, d_h), jnp.float32)] * 2,
  )(x, w_lin, b_lin, wl1, bl1, wr1)


def _stage_mid(a0, a1, c0, c1, r1, wl2, bl2, wr2, blk):
  nr, d_h = a0.shape
  d_out = wl2.shape[0]
  grid = nr // blk
  full = lambda r, c: pl.BlockSpec((r, c), lambda i: (0, 0))
  row = lambda c: pl.BlockSpec((blk, c), lambda i: (i, 0))

  def body(a0_r, a1_r, c0_r, c1_r, r1_r, wl2_r, bl2_r, wr2_r, g2_r, r2_r):
    cnt = c0_r[...] + c1_r[...]
    inv = 1.0 / jnp.maximum(cnt, 1.0)
    h1 = jnp.maximum((a0_r[...] + a1_r[...]) * inv + r1_r[...], 0.0)
    g2 = _dg(h1, wl2_r[...])
    g2_r[...] = jnp.concatenate(
        [g2, jnp.zeros((g2.shape[0], d_h - d_out), jnp.float32)], axis=1)
    r2_r[...] = _dg(h1, wr2_r[...]) + bl2_r[...]

  return pl.pallas_call(
      body,
      grid=(grid,),
      in_specs=[
          row(d_h), row(d_h), row(1), row(1), row(d_h),
          full(d_out, d_h), full(1, d_out), full(d_out, d_h),
      ],
      out_specs=[row(d_h), row(d_out)],
      out_shape=[jax.ShapeDtypeStruct((nr, d_h), jnp.float32),
                 jax.ShapeDtypeStruct((nr, d_out), jnp.float32)],
  )(a0, a1, c0, c1, r1, wl2, bl2, wr2)


def _stage_post(a0, a1, c0, c1, r2, blk):
  nr, d_pad = a0.shape
  d_out = r2.shape[1]
  grid = nr // blk
  row = lambda c: pl.BlockSpec((blk, c), lambda i: (i, 0))

  def body(a0_r, a1_r, c0_r, c1_r, r2_r, out_r):
    cnt = c0_r[...] + c1_r[...]
    inv = 1.0 / jnp.maximum(cnt, 1.0)
    agg = a0_r[...][:, :d_out] + a1_r[...][:, :d_out]
    h2 = agg * inv + r2_r[...]
    m = jnp.max(h2, axis=1, keepdims=True)
    e = jnp.exp(h2 - m)
    s = jnp.sum(e, axis=1, keepdims=True)
    out_r[...] = h2 - m - jnp.log(s)

  return pl.pallas_call(
      body,
      grid=(grid,),
      in_specs=[row(d_pad), row(d_pad), row(1), row(1), row(d_out)],
      out_specs=row(d_out),
      out_shape=jax.ShapeDtypeStruct((nr, d_out), jnp.float32),
  )(a0, a1, c0, c1, r2)


@jax.jit
def kernel(x, edge_index, W_lin, b_lin, Wl1, bl1, Wr1, Wl2, bl2, Wr2):
  n, d_in = x.shape
  d_h = W_lin.shape[0]
  d_out = Wl2.shape[0]
  e = edge_index.shape[1]

  nr = -(-n // (NS * C)) * NS * C
  n_chunks = -(-e // (NW * C))
  e_pad = NW * n_chunks * C
  pad = e_pad - e

  ar = jnp.arange(pad, dtype=jnp.int32)
  src_p = jnp.concatenate([edge_index[0], ar % jnp.int32(n)])
  dst_p = jnp.concatenate([edge_index[1], jnp.int32(n) + ar % jnp.int32(nr - n)])

  src_p = src_p.reshape(NW, n_chunks, C)
  dst_p = dst_p.reshape(NW, n_chunks, C)

  x_p = jnp.concatenate([x, jnp.zeros((nr - n, d_in), x.dtype)], axis=0)

  zeros_h = jnp.zeros((C, d_h), jnp.float32)
  ones_h = jnp.ones((C, d_h), jnp.float32)

  cnt = _make_count(n_chunks, nr, d_h)(dst_p, zeros_h, ones_h)
  c0 = cnt[0, :, 0:1]
  c1 = cnt[1, :, 0:1]

  g1, r1 = _stage_pre(x_p, W_lin, b_lin.reshape(1, -1),
                      Wl1, bl1.reshape(1, -1), Wr1, blk=1280)

  agg1 = _make_scatter(n_chunks, d_h, nr)(g1, src_p, dst_p, zeros_h)

  g2, r2 = _stage_mid(agg1[0], agg1[1], c0, c1, r1,
                      Wl2, bl2.reshape(1, -1), Wr2, blk=1280)

  agg2 = _make_scatter(n_chunks, d_h, nr)(g2, src_p, dst_p, zeros_h)

  out = _stage_post(agg2[0], agg2[1], c0, c1, r2, blk=1280)
  return out[:n]


# speedup vs baseline: 8.8015x; 1.2013x over previous
import functools

import jax
import jax.numpy as jnp
from jax import lax
from jax.experimental import pallas as pl
from jax.experimental.pallas import tpu as pltpu
from jax.experimental.pallas import tpu_sc as plsc

NC = 2
NS = 16
NW = NC * NS
LANES = 16
C = 128


def _dg(a, b):
  return lax.dot_general(a, b, (((1,), (1,)), ((), ())),
                         preferred_element_type=jnp.float32)


@functools.lru_cache(maxsize=None)
def _make_scatter(n_chunks, d, nr):
  rpt = nr // NS
  H = C // 2
  n_half = n_chunks * 2
  n_out_blk = rpt // H

  mesh = plsc.VectorSubcoreMesh(core_axis_name="c", subcore_axis_name="s")
  out_type = jax.ShapeDtypeStruct((NC, nr, d), jnp.float32)
  scratch = [
      pltpu.VMEM((n_chunks, C), jnp.int32),
      pltpu.VMEM((n_chunks, C), jnp.int32),
      pltpu.VMEM((H, d), jnp.float32),
      pltpu.VMEM((H, d), jnp.float32),
      pltpu.VMEM_SHARED((nr, d), jnp.float32),
      pltpu.SemaphoreType.DMA,
      pltpu.SemaphoreType.DMA,
  ]

  def body(table, srcs, dsts, zeros, out_acc, srcv, dstv, b0, b1, acc_sh,
           sem0, sem1):
    cid = lax.axis_index("c")
    sid = lax.axis_index("s")
    wid = sid * NC + cid

    pltpu.sync_copy(srcs.at[wid], srcv)
    pltpu.sync_copy(dsts.at[wid], dstv)
    pltpu.sync_copy(zeros.at[pl.ds(0, H)], b0)
    for b in range(n_out_blk):
      pltpu.sync_copy(b0, acc_sh.at[pl.ds(sid * rpt + b * H, H)])

    plsc.subcore_barrier()

    bufs = (b0, b1)
    sems = (sem0, sem1)

    def sidx(j):
      return srcv.at[j // 2, pl.ds((j % 2) * H, H)]

    def didx(j):
      return dstv.at[j // 2, pl.ds((j % 2) * H, H)]

    cps = [None, None]
    cps[0] = pltpu.async_copy(table.at[sidx(0)], bufs[0], sems[0])
    for j in range(n_half):
      p = j & 1
      if j + 1 < n_half:
        q = 1 - p
        cps[q] = pltpu.async_copy(table.at[sidx(j + 1)], bufs[q], sems[q])
      cps[p].wait()
      pltpu.sync_copy(bufs[p], acc_sh.at[didx(j)], add=True)

    plsc.subcore_barrier()

    for b in range(n_out_blk):
      sl = pl.ds(sid * rpt + b * H, H)
      pltpu.sync_copy(acc_sh.at[sl], b0)
      pltpu.sync_copy(b0, out_acc.at[cid, sl])

  return pl.kernel(body, out_type=out_type, mesh=mesh, scratch_types=scratch)


@functools.lru_cache(maxsize=None)
def _make_count(n_chunks, nr, w):
  rpt = nr // NS
  n_out_blk = rpt // C

  mesh = plsc.VectorSubcoreMesh(core_axis_name="c", subcore_axis_name="s")
  out_type = jax.ShapeDtypeStruct((NC, nr, w), jnp.float32)
  scratch = [
      pltpu.VMEM((n_chunks, C), jnp.int32),
      pltpu.VMEM((C, w), jnp.float32),
      pltpu.VMEM((C, w), jnp.float32),
      pltpu.VMEM_SHARED((nr, w), jnp.float32),
  ]

  def body(dsts, zeros, ones, out_cnt, dstv, ov, zc, cnt_sh):
    cid = lax.axis_index("c")
    sid = lax.axis_index("s")
    wid = sid * NC + cid

    pltpu.sync_copy(dsts.at[wid], dstv)
    pltpu.sync_copy(ones, ov)
    pltpu.sync_copy(zeros, zc)
    for b in range(n_out_blk):
      pltpu.sync_copy(zc, cnt_sh.at[pl.ds(sid * rpt + b * C, C)])

    plsc.subcore_barrier()

    def chunk(j, carry):
      pltpu.sync_copy(ov, cnt_sh.at[dstv.at[j]], add=True)
      return carry

    lax.fori_loop(0, n_chunks, chunk, 0)

    plsc.subcore_barrier()

    for b in range(n_out_blk):
      sl = pl.ds(sid * rpt + b * C, C)
      pltpu.sync_copy(cnt_sh.at[sl], zc)
      pltpu.sync_copy(zc, out_cnt.at[cid, sl])

  return pl.kernel(body, out_type=out_type, mesh=mesh, scratch_types=scratch)


def _stage_pre(x, w_lin, b_lin, wl1, bl1, wr1, blk):
  nr, d_in = x.shape
  d_h = w_lin.shape[0]
  grid = nr // blk
  full = lambda r, c: pl.BlockSpec((r, c), lambda i: (0, 0))

  def body(x_r, wlin_r, blin_r, wl1_r, bl1_r, wr1_r, g1_r, r1_r):
    h = jnp.maximum(_dg(x_r[...], wlin_r[...]) + blin_r[...], 0.0)
    g1_r[...] = _dg(h, wl1_r[...])
    r1_r[...] = _dg(h, wr1_r[...]) + bl1_r[...]

  return pl.pallas_call(
      body,
      grid=(grid,),
      in_specs=[
          pl.BlockSpec((blk, d_in), lambda i: (i, 0)),
          full(d_h, d_in), full(1, d_h),
          full(d_h, d_h), full(1, d_h), full(d_h, d_h),
      ],
      out_specs=[pl.BlockSpec((blk, d_h), lambda i: (i, 0))] * 2,
      out_shape=[jax.ShapeDtypeStruct((nr, d_h), jnp.float32)] * 2,
  )(x, w_lin, b_lin, wl1, bl1, wr1)


def _stage_mid(a0, a1, c0, c1, r1, wl2, bl2, wr2, blk):
  nr, d_h = a0.shape
  d_out = wl2.shape[0]
  grid = nr // blk
  full = lambda r, c: pl.BlockSpec((r, c), lambda i: (0, 0))
  row = lambda c: pl.BlockSpec((blk, c), lambda i: (i, 0))

  def body(a0_r, a1_r, c0_r, c1_r, r1_r, wl2_r, bl2_r, wr2_r, g2_r, r2_r):
    cnt = c0_r[...] + c1_r[...]
    inv = 1.0 / jnp.maximum(cnt, 1.0)
    h1 = jnp.maximum((a0_r[...] + a1_r[...]) * inv + r1_r[...], 0.0)
    g2 = _dg(h1, wl2_r[...])
    g2_r[...] = jnp.concatenate(
        [g2, jnp.zeros((g2.shape[0], d_h - d_out), jnp.float32)], axis=1)
    r2_r[...] = _dg(h1, wr2_r[...]) + bl2_r[...]

  return pl.pallas_call(
      body,
      grid=(grid,),
      in_specs=[
          row(d_h), row(d_h), row(1), row(1), row(d_h),
          full(d_out, d_h), full(1, d_out), full(d_out, d_h),
      ],
      out_specs=[row(d_h), row(d_out)],
      out_shape=[jax.ShapeDtypeStruct((nr, d_h), jnp.float32),
                 jax.ShapeDtypeStruct((nr, d_out), jnp.float32)],
  )(a0, a1, c0, c1, r1, wl2, bl2, wr2)


def _stage_post(a0, a1, c0, c1, r2, blk):
  nr, d_pad = a0.shape
  d_out = r2.shape[1]
  grid = nr // blk
  row = lambda c: pl.BlockSpec((blk, c), lambda i: (i, 0))

  def body(a0_r, a1_r, c0_r, c1_r, r2_r, out_r):
    cnt = c0_r[...] + c1_r[...]
    inv = 1.0 / jnp.maximum(cnt, 1.0)
    agg = a0_r[...][:, :d_out] + a1_r[...][:, :d_out]
    h2 = agg * inv + r2_r[...]
    m = jnp.max(h2, axis=1, keepdims=True)
    e = jnp.exp(h2 - m)
    s = jnp.sum(e, axis=1, keepdims=True)
    out_r[...] = h2 - m - jnp.log(s)

  return pl.pallas_call(
      body,
      grid=(grid,),
      in_specs=[row(d_pad), row(d_pad), row(1), row(1), row(d_out)],
      out_specs=row(d_out),
      out_shape=jax.ShapeDtypeStruct((nr, d_out), jnp.float32),
  )(a0, a1, c0, c1, r2)


@jax.jit
def kernel(x, edge_index, W_lin, b_lin, Wl1, bl1, Wr1, Wl2, bl2, Wr2):
  n, d_in = x.shape
  d_h = W_lin.shape[0]
  d_out = Wl2.shape[0]
  e = edge_index.shape[1]

  nr = -(-n // (NS * C)) * NS * C
  n_chunks = -(-e // (NW * C))
  e_pad = NW * n_chunks * C
  pad = e_pad - e

  ar = jnp.arange(pad, dtype=jnp.int32)
  src_p = jnp.concatenate([edge_index[0], ar % jnp.int32(n)])
  dst_p = jnp.concatenate([edge_index[1], jnp.int32(n) + ar % jnp.int32(nr - n)])

  src_p = src_p.reshape(NW, n_chunks, C)
  dst_p = dst_p.reshape(NW, n_chunks, C)

  x_p = jnp.concatenate([x, jnp.zeros((nr - n, d_in), x.dtype)], axis=0)

  zeros_h = jnp.zeros((C, d_h), jnp.float32)
  ones_h = jnp.ones((C, d_h), jnp.float32)

  cnt = _make_count(n_chunks, nr, d_h)(dst_p, zeros_h, ones_h)
  c0 = cnt[0, :, 0:1]
  c1 = cnt[1, :, 0:1]

  g1, r1 = _stage_pre(x_p, W_lin, b_lin.reshape(1, -1),
                      Wl1, bl1.reshape(1, -1), Wr1, blk=1280)

  agg1 = _make_scatter(n_chunks, d_h, nr)(g1, src_p, dst_p, zeros_h)

  g2, r2 = _stage_mid(agg1[0], agg1[1], c0, c1, r1,
                      Wl2, bl2.reshape(1, -1), Wr2, blk=1280)

  agg2 = _make_scatter(n_chunks, d_h, nr)(g2, src_p, dst_p, zeros_h)

  out = _stage_post(agg2[0], agg2[1], c0, c1, r2, blk=1280)
  return out[:n]


# speedup vs baseline: 9.3571x; 1.0631x over previous
import functools

import jax
import jax.numpy as jnp
from jax import lax
from jax.experimental import pallas as pl
from jax.experimental.pallas import tpu as pltpu
from jax.experimental.pallas import tpu_sc as plsc

NC = 2
NS = 16
NW = NC * NS
LANES = 16
C = 128


def _dg(a, b):
  return lax.dot_general(a, b, (((1,), (1,)), ((), ())),
                         preferred_element_type=jnp.float32)


@functools.lru_cache(maxsize=None)
def _make_scatter(n_chunks, d, nr):
  rpt = nr // NS
  NB = 4
  H = C // NB
  nq = n_chunks * NB
  n_out_blk = rpt // C

  mesh = plsc.VectorSubcoreMesh(core_axis_name="c", subcore_axis_name="s")
  out_type = jax.ShapeDtypeStruct((NC, nr, d), jnp.float32)
  scratch = [
      pltpu.VMEM((n_chunks, C), jnp.int32),
      pltpu.VMEM((n_chunks, C), jnp.int32),
  ] + [pltpu.VMEM((H, d), jnp.float32)] * NB + [
      pltpu.VMEM_SHARED((nr, d), jnp.float32),
  ] + [pltpu.SemaphoreType.DMA] * NB

  def body(table, srcs, dsts, zeros, out_acc, srcv, dstv, *rest):
    bufs = rest[:NB]
    acc_sh = rest[NB]
    sems = rest[NB + 1:]
    cid = lax.axis_index("c")
    sid = lax.axis_index("s")
    wid = sid * NC + cid

    pltpu.sync_copy(srcs.at[wid], srcv)
    pltpu.sync_copy(dsts.at[wid], dstv)
    for k in range(NB):
      pltpu.sync_copy(zeros.at[pl.ds(k * H, H)], bufs[k])
    for b in range(n_out_blk):
      for k in range(NB):
        pltpu.sync_copy(
            bufs[k], acc_sh.at[pl.ds(sid * rpt + b * C + k * H, H)])

    plsc.subcore_barrier()

    def sidx(j):
      return srcv.at[j // NB, pl.ds((j % NB) * H, H)]

    def didx(j):
      return dstv.at[j // NB, pl.ds((j % NB) * H, H)]

    cps = [None] * NB
    for k in range(NB - 1):
      cps[k] = pltpu.async_copy(table.at[sidx(k)], bufs[k], sems[k])
    for j in range(nq):
      p = j % NB
      if j + NB - 1 < nq:
        q = (j + NB - 1) % NB
        cps[q] = pltpu.async_copy(table.at[sidx(j + NB - 1)], bufs[q], sems[q])
      cps[p].wait()
      pltpu.sync_copy(bufs[p], acc_sh.at[didx(j)], add=True)

    plsc.subcore_barrier()

    for b in range(n_out_blk):
      for k in range(NB):
        sl = pl.ds(sid * rpt + b * C + k * H, H)
        pltpu.sync_copy(acc_sh.at[sl], bufs[k])
        pltpu.sync_copy(bufs[k], out_acc.at[cid, sl])

  return pl.kernel(body, out_type=out_type, mesh=mesh, scratch_types=scratch)


@functools.lru_cache(maxsize=None)
def _make_count(n_chunks, nr, w):
  rpt = nr // NS
  n_out_blk = rpt // C

  mesh = plsc.VectorSubcoreMesh(core_axis_name="c", subcore_axis_name="s")
  out_type = jax.ShapeDtypeStruct((NC, nr, w), jnp.float32)
  scratch = [
      pltpu.VMEM((n_chunks, C), jnp.int32),
      pltpu.VMEM((C, w), jnp.float32),
      pltpu.VMEM((C, w), jnp.float32),
      pltpu.VMEM_SHARED((nr, w), jnp.float32),
  ]

  def body(dsts, zeros, ones, out_cnt, dstv, ov, zc, cnt_sh):
    cid = lax.axis_index("c")
    sid = lax.axis_index("s")
    wid = sid * NC + cid

    pltpu.sync_copy(dsts.at[wid], dstv)
    pltpu.sync_copy(ones, ov)
    pltpu.sync_copy(zeros, zc)
    for b in range(n_out_blk):
      pltpu.sync_copy(zc, cnt_sh.at[pl.ds(sid * rpt + b * C, C)])

    plsc.subcore_barrier()

    def chunk(j, carry):
      pltpu.sync_copy(ov, cnt_sh.at[dstv.at[j]], add=True)
      return carry

    lax.fori_loop(0, n_chunks, chunk, 0)

    plsc.subcore_barrier()

    for b in range(n_out_blk):
      sl = pl.ds(sid * rpt + b * C, C)
      pltpu.sync_copy(cnt_sh.at[sl], zc)
      pltpu.sync_copy(zc, out_cnt.at[cid, sl])

  return pl.kernel(body, out_type=out_type, mesh=mesh, scratch_types=scratch)


def _stage_pre(x, w_lin, b_lin, wl1, bl1, wr1, blk):
  nr, d_in = x.shape
  d_h = w_lin.shape[0]
  grid = nr // blk
  full = lambda r, c: pl.BlockSpec((r, c), lambda i: (0, 0))

  def body(x_r, wlin_r, blin_r, wl1_r, bl1_r, wr1_r, g1_r, r1_r):
    h = jnp.maximum(_dg(x_r[...], wlin_r[...]) + blin_r[...], 0.0)
    g1_r[...] = _dg(h, wl1_r[...])
    r1_r[...] = _dg(h, wr1_r[...]) + bl1_r[...]

  return pl.pallas_call(
      body,
      grid=(grid,),
      in_specs=[
          pl.BlockSpec((blk, d_in), lambda i: (i, 0)),
          full(d_h, d_in), full(1, d_h),
          full(d_h, d_h), full(1, d_h), full(d_h, d_h),
      ],
      out_specs=[pl.BlockSpec((blk, d_h), lambda i: (i, 0))] * 2,
      out_shape=[jax.ShapeDtypeStruct((nr, d_h), jnp.float32)] * 2,
  )(x, w_lin, b_lin, wl1, bl1, wr1)


def _stage_mid(a0, a1, c0, c1, r1, wl2, bl2, wr2, blk):
  nr, d_h = a0.shape
  d_out = wl2.shape[0]
  grid = nr // blk
  full = lambda r, c: pl.BlockSpec((r, c), lambda i: (0, 0))
  row = lambda c: pl.BlockSpec((blk, c), lambda i: (i, 0))

  def body(a0_r, a1_r, c0_r, c1_r, r1_r, wl2_r, bl2_r, wr2_r, g2_r, r2_r):
    cnt = c0_r[...] + c1_r[...]
    inv = 1.0 / jnp.maximum(cnt, 1.0)
    h1 = jnp.maximum((a0_r[...] + a1_r[...]) * inv + r1_r[...], 0.0)
    g2 = _dg(h1, wl2_r[...])
    g2_r[...] = jnp.concatenate(
        [g2, jnp.zeros((g2.shape[0], d_h - d_out), jnp.float32)], axis=1)
    r2_r[...] = _dg(h1, wr2_r[...]) + bl2_r[...]

  return pl.pallas_call(
      body,
      grid=(grid,),
      in_specs=[
          row(d_h), row(d_h), row(1), row(1), row(d_h),
          full(d_out, d_h), full(1, d_out), full(d_out, d_h),
      ],
      out_specs=[row(d_h), row(d_out)],
      out_shape=[jax.ShapeDtypeStruct((nr, d_h), jnp.float32),
                 jax.ShapeDtypeStruct((nr, d_out), jnp.float32)],
  )(a0, a1, c0, c1, r1, wl2, bl2, wr2)


def _stage_post(a0, a1, c0, c1, r2, blk):
  nr, d_pad = a0.shape
  d_out = r2.shape[1]
  grid = nr // blk
  row = lambda c: pl.BlockSpec((blk, c), lambda i: (i, 0))

  def body(a0_r, a1_r, c0_r, c1_r, r2_r, out_r):
    cnt = c0_r[...] + c1_r[...]
    inv = 1.0 / jnp.maximum(cnt, 1.0)
    agg = a0_r[...][:, :d_out] + a1_r[...][:, :d_out]
    h2 = agg * inv + r2_r[...]
    m = jnp.max(h2, axis=1, keepdims=True)
    e = jnp.exp(h2 - m)
    s = jnp.sum(e, axis=1, keepdims=True)
    out_r[...] = h2 - m - jnp.log(s)

  return pl.pallas_call(
      body,
      grid=(grid,),
      in_specs=[row(d_pad), row(d_pad), row(1), row(1), row(d_out)],
      out_specs=row(d_out),
      out_shape=jax.ShapeDtypeStruct((nr, d_out), jnp.float32),
  )(a0, a1, c0, c1, r2)


@jax.jit
def kernel(x, edge_index, W_lin, b_lin, Wl1, bl1, Wr1, Wl2, bl2, Wr2):
  n, d_in = x.shape
  d_h = W_lin.shape[0]
  d_out = Wl2.shape[0]
  e = edge_index.shape[1]

  nr = -(-n // (NS * C)) * NS * C
  n_chunks = -(-e // (NW * C))
  e_pad = NW * n_chunks * C
  pad = e_pad - e

  ar = jnp.arange(pad, dtype=jnp.int32)
  src_p = jnp.concatenate([edge_index[0], ar % jnp.int32(n)])
  dst_p = jnp.concatenate([edge_index[1], jnp.int32(n) + ar % jnp.int32(nr - n)])

  src_p = src_p.reshape(NW, n_chunks, C)
  dst_p = dst_p.reshape(NW, n_chunks, C)

  x_p = jnp.concatenate([x, jnp.zeros((nr - n, d_in), x.dtype)], axis=0)

  zeros_h = jnp.zeros((C, d_h), jnp.float32)
  ones_h = jnp.ones((C, d_h), jnp.float32)

  cnt = _make_count(n_chunks, nr, d_h)(dst_p, zeros_h, ones_h)
  c0 = cnt[0, :, 0:1]
  c1 = cnt[1, :, 0:1]

  g1, r1 = _stage_pre(x_p, W_lin, b_lin.reshape(1, -1),
                      Wl1, bl1.reshape(1, -1), Wr1, blk=1280)

  agg1 = _make_scatter(n_chunks, d_h, nr)(g1, src_p, dst_p, zeros_h)

  g2, r2 = _stage_mid(agg1[0], agg1[1], c0, c1, r1,
                      Wl2, bl2.reshape(1, -1), Wr2, blk=1280)

  agg2 = _make_scatter(n_chunks, d_h, nr)(g2, src_p, dst_p, zeros_h)

  out = _stage_post(agg2[0], agg2[1], c0, c1, r2, blk=1280)
  return out[:n]
